# Initial kernel scaffold; baseline (speedup 1.0000x reference)
import functools
import jax
import jax.numpy as jnp
from jax import lax
from jax.experimental import pallas as pl
from jax.experimental.pallas import tpu as pltpu
from jax.experimental.pallas import tpu_sc as plsc

NC = 2
NS = 16
L = 16
NW = NC * NS

CW = 112


@functools.partial(jax.jit, static_argnums=(3, 4, 5))
def _sc_gather(table, adjc_blk, voff, n_chunks, n_var, C):
    bpad = NW * n_chunks * CW
    rpw = n_chunks * CW
    mesh = plsc.VectorSubcoreMesh(
        core_axis_name="c", subcore_axis_name="s", num_cores=NC, num_subcores=NS
    )

    @functools.partial(
        pl.kernel,
        out_type=jax.ShapeDtypeStruct((n_var, bpad, C), jnp.float32),
        mesh=mesh,
        scratch_types=[
            pltpu.VMEM((n_chunks, CW), jnp.int32),
            pltpu.VMEM((n_chunks, CW), jnp.int32),
            pltpu.VMEM((n_var, L), jnp.int32),
            pltpu.VMEM((2, CW, C), jnp.float32),
            pltpu.SemaphoreType.DMA,
            pltpu.SemaphoreType.DMA,
        ],
    )
    def k(table_h, adjc_h, voff_h, out_h, idx_raw, idx_adj, voff_v, rows, s0, s1):
        wid = lax.axis_index("s") * NC + lax.axis_index("c")
        pltpu.sync_copy(adjc_h.at[pl.ds(wid * n_chunks, n_chunks), :], idx_raw)
        pltpu.sync_copy(voff_h, voff_v)
        sems = (s0, s1)
        for v in range(n_var):
            off = voff_v[v, :]
            for j in range(n_chunks):
                for i in range(CW // L):
                    sl = pl.ds(i * L, L)
                    idx_adj[j, sl] = idx_raw[j, sl] + off
            handles = [None, None]
            handles[0] = pltpu.async_copy(
                table_h.at[idx_adj.at[0]], rows.at[0], sems[0]
            )
            for j in range(n_chunks):
                b = j % 2
                if j + 1 < n_chunks:
                    nb = (j + 1) % 2
                    handles[nb] = pltpu.async_copy(
                        table_h.at[idx_adj.at[j + 1]], rows.at[nb], sems[nb]
                    )
                handles[b].wait()
                pltpu.sync_copy(
                    rows.at[b],
                    out_h.at[v, pl.ds(wid * rpw + j * CW, CW), :],
                )

    return k(table, adjc_blk, voff)


def kernel(mg_emb, var_indices, adjc):
    n_vars, T, S, C = mg_emb.shape
    B, V = var_indices.shape
    n_var = B * V

    chunk_rows = NW * CW
    n_chunks = -(-S // chunk_rows)
    bpad = NW * n_chunks * CW

    table = mg_emb.reshape(n_vars * T * S, C)
    col = adjc[:, 0].astype(jnp.int32)
    col = jnp.pad(col, (0, bpad - S))
    adjc_blk = col.reshape(NW * n_chunks, CW)
    vi = var_indices.reshape(-1).astype(jnp.int32)
    voff = jnp.broadcast_to((vi * (T * S))[:, None], (n_var, L))

    out = _sc_gather(table, adjc_blk, voff, n_chunks, n_var, C)
    return out[:, :S, :].reshape(B, V, T, S, C)


# speedup vs baseline: 9.6312x; 9.6312x over previous
import functools
import jax
import jax.numpy as jnp
from jax import lax
from jax.experimental import pallas as pl
from jax.experimental.pallas import tpu as pltpu
from jax.experimental.pallas import tpu_sc as plsc

NC = 2
NS = 16
L = 16
NW = NC * NS

CW = 112


@functools.partial(jax.jit, static_argnums=(3, 4, 5))
def _sc_gather(table, adjc_blk, voff, n_chunks, n_var, C):
    bpad = NW * n_chunks * CW
    rpw = n_chunks * CW
    mesh = plsc.VectorSubcoreMesh(
        core_axis_name="c", subcore_axis_name="s", num_cores=NC, num_subcores=NS
    )

    @functools.partial(
        pl.kernel,
        out_type=jax.ShapeDtypeStruct((n_var, bpad, C), jnp.float32),
        mesh=mesh,
        scratch_types=[
            pltpu.VMEM((n_chunks, CW), jnp.int32),
            pltpu.VMEM((n_chunks, CW), jnp.int32),
            pltpu.VMEM((n_var, L), jnp.int32),
            pltpu.VMEM((2, CW, C), jnp.float32),
            pltpu.SemaphoreType.DMA,
            pltpu.SemaphoreType.DMA,
        ],
    )
    def k(table_h, adjc_h, voff_h, out_h, idx_raw, idx_adj, voff_v, rows, s0, s1):
        wid = lax.axis_index("s") * NC + lax.axis_index("c")
        pltpu.sync_copy(adjc_h.at[wid], idx_raw)
        pltpu.sync_copy(voff_h, voff_v)
        sems = (s0, s1)
        for v in range(n_var):
            off = voff_v[v, :]
            for j in range(n_chunks):
                for i in range(CW // L):
                    sl = pl.ds(i * L, L)
                    idx_adj[j, sl] = idx_raw[j, sl] + off
            handles = [None, None]
            handles[0] = pltpu.async_copy(
                table_h.at[idx_adj.at[0]], rows.at[0], sems[0]
            )
            for j in range(n_chunks):
                b = j % 2
                if j + 1 < n_chunks:
                    nb = (j + 1) % 2
                    handles[nb] = pltpu.async_copy(
                        table_h.at[idx_adj.at[j + 1]], rows.at[nb], sems[nb]
                    )
                handles[b].wait()
                pltpu.sync_copy(
                    rows.at[b],
                    out_h.at[v, pl.ds(wid * rpw + j * CW, CW), :],
                )

    return k(table, adjc_blk, voff)


def kernel(mg_emb, var_indices, adjc):
    n_vars, T, S, C = mg_emb.shape
    B, V = var_indices.shape
    n_var = B * V

    chunk_rows = NW * CW
    n_chunks = -(-S // chunk_rows)
    bpad = NW * n_chunks * CW

    table = mg_emb.reshape(n_vars * T * S, C)
    col = adjc[:, 0].astype(jnp.int32)
    col = jnp.pad(col, (0, bpad - S))
    adjc_blk = col.reshape(NW, n_chunks, CW)
    vi = var_indices.reshape(-1).astype(jnp.int32)
    voff = jnp.broadcast_to((vi * (T * S))[:, None], (n_var, L))

    out = _sc_gather(table, adjc_blk, voff, n_chunks, n_var, C)
    return out[:, :S, :].reshape(B, V, T, S, C)


# speedup vs baseline: 10.2332x; 1.0625x over previous
import functools
import jax
import jax.numpy as jnp
from jax import lax
from jax.experimental import pallas as pl
from jax.experimental.pallas import tpu as pltpu
from jax.experimental.pallas import tpu_sc as plsc

NC = 2
NS = 16
L = 16
NW = NC * NS

CW = 112


@functools.partial(jax.jit, static_argnums=(3, 4, 5))
def _sc_gather(table, adjc_blk, voff, n_chunks, n_var, C):
    bpad = NW * n_chunks * CW
    rpw = n_chunks * CW
    mesh = plsc.VectorSubcoreMesh(
        core_axis_name="c", subcore_axis_name="s", num_cores=NC, num_subcores=NS
    )

    total = n_var * n_chunks
    nbuf = min(4, total)

    @functools.partial(
        pl.kernel,
        out_type=jax.ShapeDtypeStruct((n_var, bpad, C), jnp.float32),
        mesh=mesh,
        scratch_types=[
            pltpu.VMEM((n_chunks, CW), jnp.int32),
            pltpu.VMEM((total, CW), jnp.int32),
            pltpu.VMEM((n_var, L), jnp.int32),
            pltpu.VMEM((nbuf, CW, C), jnp.float32),
            [pltpu.SemaphoreType.DMA] * nbuf,
            [pltpu.SemaphoreType.DMA] * nbuf,
        ],
    )
    def k(table_h, adjc_h, voff_h, out_h, idx_raw, idx_adj, voff_v, rows, gsem, wsem):
        wid = lax.axis_index("s") * NC + lax.axis_index("c")
        pltpu.sync_copy(adjc_h.at[wid], idx_raw)
        pltpu.sync_copy(voff_h, voff_v)
        for v in range(n_var):
            off = voff_v[v, :]
            for j in range(n_chunks):
                for i in range(CW // L):
                    sl = pl.ds(i * L, L)
                    idx_adj[v * n_chunks + j, sl] = idx_raw[j, sl] + off
        g = [None] * nbuf
        w = [None] * nbuf
        for t in range(nbuf):
            g[t] = pltpu.async_copy(table_h.at[idx_adj.at[t]], rows.at[t], gsem[t])
        for t in range(total):
            b = t % nbuf
            v, j = divmod(t, n_chunks)
            g[b].wait()
            w[b] = pltpu.async_copy(
                rows.at[b], out_h.at[v, pl.ds(wid * rpw + j * CW, CW), :], wsem[b]
            )
            f = t + nbuf
            w[b].wait()
            if f < total:
                g[b] = pltpu.async_copy(
                    table_h.at[idx_adj.at[f]], rows.at[b], gsem[b]
                )

    return k(table, adjc_blk, voff)


def kernel(mg_emb, var_indices, adjc):
    n_vars, T, S, C = mg_emb.shape
    B, V = var_indices.shape
    n_var = B * V

    chunk_rows = NW * CW
    n_chunks = -(-S // chunk_rows)
    bpad = NW * n_chunks * CW

    table = mg_emb.reshape(n_vars * T * S, C)
    col = adjc[:, 0].astype(jnp.int32)
    col = jnp.pad(col, (0, bpad - S))
    adjc_blk = col.reshape(NW, n_chunks, CW)
    vi = var_indices.reshape(-1).astype(jnp.int32)
    voff = jnp.broadcast_to((vi * (T * S))[:, None], (n_var, L))

    out = _sc_gather(table, adjc_blk, voff, n_chunks, n_var, C)
    return out[:, :S, :].reshape(B, V, T, S, C)


# speedup vs baseline: 16.5586x; 1.6181x over previous
import functools
import jax
import jax.numpy as jnp
import numpy as np
from jax import lax
from jax.experimental import pallas as pl
from jax.experimental.pallas import tpu as pltpu
from jax.experimental.pallas import tpu_sc as plsc

NC = 2
NS = 16
L = 16
NW = NC * NS

CW = 112


@functools.partial(jax.jit, static_argnums=(3, 4, 5, 6))
def _sc_gather(table, adjc_blk, voff, n_chunks, n_var, C, S):
    rpw = n_chunks * CW
    mesh = plsc.VectorSubcoreMesh(
        core_axis_name="c", subcore_axis_name="s", num_cores=NC, num_subcores=NS
    )

    total = n_var * n_chunks
    nbuf = min(4, total)

    @functools.partial(
        pl.kernel,
        out_type=jax.ShapeDtypeStruct((n_var, S, C), jnp.float32),
        mesh=mesh,
        scratch_types=[
            pltpu.VMEM((n_chunks, CW), jnp.int32),
            pltpu.VMEM((total, CW), jnp.int32),
            pltpu.VMEM((n_var, L), jnp.int32),
            pltpu.VMEM((nbuf, CW, C), jnp.float32),
            [pltpu.SemaphoreType.DMA] * nbuf,
            [pltpu.SemaphoreType.DMA] * nbuf,
        ],
    )
    def k(table_h, adjc_h, voff_h, out_h, idx_raw, idx_adj, voff_v, rows, gsem, wsem):
        wid = lax.axis_index("s") * NC + lax.axis_index("c")
        start = lax.min(wid * rpw, S - rpw)
        pltpu.sync_copy(adjc_h.at[wid], idx_raw)
        pltpu.sync_copy(voff_h, voff_v)
        for v in range(n_var):
            off = voff_v[v, :]
            for j in range(n_chunks):
                for i in range(CW // L):
                    sl = pl.ds(i * L, L)
                    idx_adj[v * n_chunks + j, sl] = idx_raw[j, sl] + off
        g = [None] * nbuf
        w = [None] * nbuf
        for t in range(nbuf):
            g[t] = pltpu.async_copy(table_h.at[idx_adj.at[t]], rows.at[t], gsem[t])
        for t in range(total):
            b = t % nbuf
            v, j = divmod(t, n_chunks)
            g[b].wait()
            w[b] = pltpu.async_copy(
                rows.at[b], out_h.at[v, pl.ds(start + j * CW, CW), :], wsem[b]
            )
            f = t + nbuf
            w[b].wait()
            if f < total:
                g[b] = pltpu.async_copy(
                    table_h.at[idx_adj.at[f]], rows.at[b], gsem[b]
                )

    return k(table, adjc_blk, voff)


def kernel(mg_emb, var_indices, adjc):
    n_vars, T, S, C = mg_emb.shape
    B, V = var_indices.shape
    n_var = B * V

    chunk_rows = NW * CW
    n_chunks = -(-S // chunk_rows)
    rpw = n_chunks * CW

    table = mg_emb.reshape(n_vars * T * S, C)
    col = adjc[:, 0].astype(jnp.int32)
    starts = np.minimum(np.arange(NW) * rpw, S - rpw)
    windows = starts[:, None] + np.arange(rpw)[None, :]
    adjc_blk = col[windows].reshape(NW, n_chunks, CW)
    vi = var_indices.reshape(-1).astype(jnp.int32)
    voff = jnp.broadcast_to((vi * (T * S))[:, None], (n_var, L))

    out = _sc_gather(table, adjc_blk, voff, n_chunks, n_var, C, S)
    return out.reshape(B, V, T, S, C)


# speedup vs baseline: 19.4767x; 1.1762x over previous
import functools
import jax
import jax.numpy as jnp
import numpy as np
from jax import lax
from jax.experimental import pallas as pl
from jax.experimental.pallas import tpu as pltpu
from jax.experimental.pallas import tpu_sc as plsc

NC = 2
NS = 16
L = 16
NW = NC * NS

CW = 112


@functools.partial(jax.jit, static_argnums=(3, 4, 5, 6))
def _sc_gather(table, col, voff, n_chunks, n_var, C, S):
    rpw = n_chunks * CW
    mesh = plsc.VectorSubcoreMesh(
        core_axis_name="c", subcore_axis_name="s", num_cores=NC, num_subcores=NS
    )

    total = n_var * n_chunks
    nbuf = min(4, total)

    @functools.partial(
        pl.kernel,
        out_type=jax.ShapeDtypeStruct((n_var, S, C), jnp.float32),
        mesh=mesh,
        scratch_types=[
            pltpu.VMEM((rpw,), jnp.int32),
            pltpu.VMEM((total, CW), jnp.int32),
            pltpu.VMEM((n_var, L), jnp.int32),
            pltpu.VMEM((nbuf, CW, C), jnp.float32),
            [pltpu.SemaphoreType.DMA] * nbuf,
            [pltpu.SemaphoreType.DMA] * nbuf,
        ],
    )
    def k(table_h, adjc_h, voff_h, out_h, idx_raw, idx_adj, voff_v, rows, gsem, wsem):
        wid = lax.axis_index("s") * NC + lax.axis_index("c")
        start = lax.min(wid * rpw, S - rpw)
        pltpu.sync_copy(adjc_h.at[pl.ds(start, rpw)], idx_raw)
        pltpu.sync_copy(voff_h, voff_v)
        for v in range(n_var):
            off = voff_v[v, :]
            for j in range(n_chunks):
                for i in range(CW // L):
                    sl = pl.ds(i * L, L)
                    idx_adj[v * n_chunks + j, sl] = idx_raw[pl.ds(j * CW + i * L, L)] + off
        g = [None] * nbuf
        w = [None] * nbuf
        for t in range(nbuf):
            g[t] = pltpu.async_copy(table_h.at[idx_adj.at[t]], rows.at[t], gsem[t])
        for t in range(total):
            b = t % nbuf
            v, j = divmod(t, n_chunks)
            g[b].wait()
            w[b] = pltpu.async_copy(
                rows.at[b], out_h.at[v, pl.ds(start + j * CW, CW), :], wsem[b]
            )
            f = t + nbuf
            w[b].wait()
            if f < total:
                g[b] = pltpu.async_copy(
                    table_h.at[idx_adj.at[f]], rows.at[b], gsem[b]
                )

    return k(table, col, voff)


def kernel(mg_emb, var_indices, adjc):
    n_vars, T, S, C = mg_emb.shape
    B, V = var_indices.shape
    n_var = B * V

    chunk_rows = NW * CW
    n_chunks = -(-S // chunk_rows)
    rpw = n_chunks * CW

    table = mg_emb.reshape(n_vars * T * S, C)
    vi = var_indices.reshape(-1).astype(jnp.int32)
    voff = jnp.broadcast_to((vi * (T * S))[:, None], (n_var, L))

    col = adjc[:, 0].astype(jnp.int32)
    out = _sc_gather(table, col, voff, n_chunks, n_var, C, S)
    return out.reshape(B, V, T, S, C)


# speedup vs baseline: 19.5871x; 1.0057x over previous
import functools
import jax
import jax.numpy as jnp
import numpy as np
from jax import lax
from jax.experimental import pallas as pl
from jax.experimental.pallas import tpu as pltpu
from jax.experimental.pallas import tpu_sc as plsc

NC = 2
NS = 16
L = 16
NW = NC * NS

CW = 112


@functools.partial(jax.jit, static_argnums=(3, 4, 5, 6))
def _sc_gather(table, col, voff, n_chunks, n_var, C, S):
    rpw = n_chunks * CW
    mesh = plsc.VectorSubcoreMesh(
        core_axis_name="c", subcore_axis_name="s", num_cores=NC, num_subcores=NS
    )

    total = n_var * n_chunks
    nbuf = min(6, total)

    @functools.partial(
        pl.kernel,
        out_type=jax.ShapeDtypeStruct((n_var, S, C), jnp.float32),
        mesh=mesh,
        scratch_types=[
            pltpu.VMEM((rpw,), jnp.int32),
            pltpu.VMEM((total, CW), jnp.int32),
            pltpu.VMEM((n_var, L), jnp.int32),
            pltpu.VMEM((nbuf, CW, C), jnp.float32),
            [pltpu.SemaphoreType.DMA] * nbuf,
            [pltpu.SemaphoreType.DMA] * nbuf,
        ],
    )
    def k(table_h, adjc_h, voff_h, out_h, idx_raw, idx_adj, voff_v, rows, gsem, wsem):
        wid = lax.axis_index("s") * NC + lax.axis_index("c")
        start = lax.min(wid * rpw, S - rpw)
        pltpu.sync_copy(adjc_h.at[pl.ds(start, rpw)], idx_raw)
        pltpu.sync_copy(voff_h, voff_v)
        for v in range(n_var):
            off = voff_v[v, :]
            for j in range(n_chunks):
                for i in range(CW // L):
                    sl = pl.ds(i * L, L)
                    idx_adj[v * n_chunks + j, sl] = idx_raw[pl.ds(j * CW + i * L, L)] + off
        g = [None] * nbuf
        w = [None] * nbuf
        for t in range(nbuf):
            g[t] = pltpu.async_copy(table_h.at[idx_adj.at[t]], rows.at[t], gsem[t])
        for t in range(total):
            b = t % nbuf
            v, j = divmod(t, n_chunks)
            g[b].wait()
            w[b] = pltpu.async_copy(
                rows.at[b], out_h.at[v, pl.ds(start + j * CW, CW), :], wsem[b]
            )
            f = t + nbuf
            w[b].wait()
            if f < total:
                g[b] = pltpu.async_copy(
                    table_h.at[idx_adj.at[f]], rows.at[b], gsem[b]
                )

    return k(table, col, voff)


def kernel(mg_emb, var_indices, adjc):
    n_vars, T, S, C = mg_emb.shape
    B, V = var_indices.shape
    n_var = B * V

    chunk_rows = NW * CW
    n_chunks = -(-S // chunk_rows)
    rpw = n_chunks * CW

    table = mg_emb.reshape(n_vars * T * S, C)
    vi = var_indices.reshape(-1).astype(jnp.int32)
    voff = jnp.broadcast_to((vi * (T * S))[:, None], (n_var, L))

    col = adjc[:, 0].astype(jnp.int32)
    out = _sc_gather(table, col, voff, n_chunks, n_var, C, S)
    return out.reshape(B, V, T, S, C)
